# Initial kernel scaffold; baseline (speedup 1.0000x reference)
import functools

import jax
import jax.numpy as jnp
from jax import lax
from jax.experimental import pallas as pl
from jax.experimental.pallas import tpu as pltpu
from jax.experimental.pallas import tpu_sc as plsc

_NC = 2
_NS = 16
_L = 16
_NW = _NC * _NS


def _sc_mesh():
    return plsc.VectorSubcoreMesh(
        core_axis_name="c", subcore_axis_name="s",
        num_cores=_NC, num_subcores=_NS)


def _deg_sc(row, n):
    e = row.shape[0]
    per = e // _NW
    assert per * _NW == e and per % _L == 0 and n % _L == 0

    @functools.partial(
        pl.kernel,
        out_type=jax.ShapeDtypeStruct((_NW, n), jnp.int32),
        mesh=_sc_mesh(),
        scratch_types=[
            pltpu.VMEM((per,), jnp.int32),
            pltpu.VMEM((n,), jnp.int32),
        ],
    )
    def deg_kernel(row_hbm, out_hbm, idx_v, deg_v):
        c = lax.axis_index("c")
        s = lax.axis_index("s")
        w = c * _NS + s
        zeros = jnp.zeros((_L,), jnp.int32)

        def zbody(k, carry):
            deg_v[pl.ds(k * _L, _L)] = zeros
            return carry
        lax.fori_loop(0, n // _L, zbody, None)

        pltpu.sync_copy(row_hbm.at[pl.ds(w * per, per)], idx_v)

        def body(k, carry):
            idx = idx_v[pl.ds(k * _L, _L)]
            cnt, last = plsc.scan_count(idx)
            plsc.addupdate_scatter(deg_v, [idx], cnt, mask=last)
            return carry
        lax.fori_loop(0, per // _L, body, None)

        pltpu.sync_copy(deg_v, out_hbm.at[w])

    return deg_kernel(row)


def _norm_body(dp_ref, norm_ref):
    s = jnp.sum(dp_ref[...].astype(jnp.float32), axis=0, keepdims=True)
    norm_ref[...] = lax.rsqrt(jnp.maximum(s, 1.0))


def _norm_tc(deg_parts):
    nw, n = deg_parts.shape
    return pl.pallas_call(
        _norm_body,
        out_shape=jax.ShapeDtypeStruct((1, n), jnp.float32),
    )(deg_parts)


def _scale_body(h_ref, n_ref, o_ref):
    o_ref[...] = h_ref[...] * n_ref[...]


def _scale_tc(h, norm_col):
    n, d = h.shape
    bn = 2000
    return pl.pallas_call(
        _scale_body,
        grid=(n // bn,),
        in_specs=[pl.BlockSpec((bn, d), lambda i: (i, 0)),
                  pl.BlockSpec((bn, 1), lambda i: (i, 0))],
        out_specs=pl.BlockSpec((bn, d), lambda i: (i, 0)),
        out_shape=jax.ShapeDtypeStruct((n, d), jnp.float32),
    )(h, norm_col)


def _spmm_sc(hn, col, row):
    n, d = hn.shape
    e = col.shape[0]
    per = e // _NW
    ch = 80
    nch = per // ch
    rpt = n // _NS
    zrows = 125
    assert per * _NW == e and nch * ch == per
    assert rpt * _NS == n and rpt % zrows == 0 and d % _L == 0

    @functools.partial(
        pl.kernel,
        out_type=jax.ShapeDtypeStruct((_NC, n, d), jnp.float32),
        mesh=_sc_mesh(),
        scratch_types=[
            pltpu.VMEM((ch,), jnp.int32),
            pltpu.VMEM((ch,), jnp.int32),
            pltpu.VMEM((ch, d), jnp.float32),
            pltpu.VMEM((zrows, d), jnp.float32),
            pltpu.VMEM_SHARED((n, d), jnp.float32),
        ],
    )
    def spmm_kernel(hn_hbm, col_hbm, row_hbm, out_hbm,
                    cidx, ridx, gbuf, zbuf, agg_s):
        c = lax.axis_index("c")
        s = lax.axis_index("s")
        w = c * _NS + s
        base = w * per
        zeros = jnp.zeros((_L,), jnp.float32)

        def zrow(r, carry):
            for q in range(d // _L):
                zbuf[r, pl.ds(q * _L, _L)] = zeros
            return carry
        lax.fori_loop(0, zrows, zrow, None)

        for k in range(rpt // zrows):
            pltpu.sync_copy(zbuf, agg_s.at[pl.ds(s * rpt + k * zrows, zrows)])
        plsc.subcore_barrier()

        def body(j, carry):
            off = base + j * ch
            pltpu.sync_copy(col_hbm.at[pl.ds(off, ch)], cidx)
            pltpu.sync_copy(hn_hbm.at[cidx], gbuf)
            pltpu.sync_copy(row_hbm.at[pl.ds(off, ch)], ridx)
            pltpu.sync_copy(gbuf, agg_s.at[ridx], add=True)
            return carry
        lax.fori_loop(0, nch, body, None)

        plsc.subcore_barrier()
        pltpu.sync_copy(agg_s.at[pl.ds(s * rpt, rpt)],
                        out_hbm.at[c].at[pl.ds(s * rpt, rpt)])

    return spmm_kernel(hn, col, row)


def _final_body(a_ref, n_ref, w_ref, o_ref):
    a = a_ref[0] + a_ref[1]
    sc = a * n_ref[...]
    o_ref[...] = lax.dot_general(
        sc, w_ref[...], (((1,), (1,)), ((), ())),
        preferred_element_type=jnp.float32)


def _final_tc(agg2, norm_col, W):
    _, n, d = agg2.shape
    bn = 2000
    return pl.pallas_call(
        _final_body,
        grid=(n // bn,),
        in_specs=[pl.BlockSpec((2, bn, d), lambda i: (0, i, 0)),
                  pl.BlockSpec((bn, 1), lambda i: (i, 0)),
                  pl.BlockSpec((d, d), lambda i: (0, 0))],
        out_specs=pl.BlockSpec((bn, d), lambda i: (i, 0)),
        out_shape=jax.ShapeDtypeStruct((n, d), jnp.float32),
    )(agg2, norm_col, W)


def kernel(edge_index, h, W):
    n, d = h.shape
    row = edge_index[0]
    col = edge_index[1]
    deg_parts = _deg_sc(row, n)
    norm = _norm_tc(deg_parts)
    norm_col = norm.reshape(n, 1)
    hn = _scale_tc(h, norm_col)
    agg2 = _spmm_sc(hn, col, row)
    return _final_tc(agg2, norm_col, W)


# speedup vs baseline: 6.3931x; 6.3931x over previous
import functools

import jax
import jax.numpy as jnp
from jax import lax
from jax.experimental import pallas as pl
from jax.experimental.pallas import tpu as pltpu
from jax.experimental.pallas import tpu_sc as plsc

_NC = 2
_NS = 16
_L = 16
_NW = _NC * _NS
_CH = 128


def _sc_mesh():
    return plsc.VectorSubcoreMesh(
        core_axis_name="c", subcore_axis_name="s",
        num_cores=_NC, num_subcores=_NS)


def _pad_n(n):
    q = _CH * _NS
    return (n + q - 1) // q * q


def _deg_sc(row, n_pad):
    e = row.shape[0]
    tch = e // _CH
    zn = n_pad // _NS
    assert tch * _CH == e and zn % _CH == 0

    @functools.partial(
        pl.kernel,
        out_type=jax.ShapeDtypeStruct((_NC, n_pad), jnp.float32),
        mesh=_sc_mesh(),
        scratch_types=[
            pltpu.VMEM((_CH,), jnp.int32),
            pltpu.VMEM((_CH,), jnp.float32),
            pltpu.VMEM((zn,), jnp.float32),
            pltpu.VMEM_SHARED((n_pad,), jnp.float32),
        ],
    )
    def deg_kernel(row_hbm, out_hbm, ridx, ones_v, zv, deg_s):
        c = lax.axis_index("c")
        s = lax.axis_index("s")
        w = c * _NS + s
        lo = w * tch // _NW
        hi = (w + 1) * tch // _NW
        ones = jnp.ones((_L,), jnp.float32)
        zeros = jnp.zeros((_L,), jnp.float32)

        for q in range(_CH // _L):
            ones_v[pl.ds(q * _L, _L)] = ones

        def zbody(k, carry):
            zv[pl.ds(k * _L, _L)] = zeros
            return carry
        lax.fori_loop(0, zn // _L, zbody, None)
        pltpu.sync_copy(zv, deg_s.at[pl.ds(s * zn, zn)])
        plsc.subcore_barrier()

        def body(j, carry):
            pltpu.sync_copy(row_hbm.at[pl.ds(j * _CH, _CH)], ridx)
            pltpu.sync_copy(ones_v, deg_s.at[ridx], add=True)
            return carry
        lax.fori_loop(lo, hi, body, None)

        plsc.subcore_barrier()
        pltpu.sync_copy(deg_s.at[pl.ds(s * zn, zn)],
                        out_hbm.at[c].at[pl.ds(s * zn, zn)])

    return deg_kernel(row)


def _norm_body(dp_ref, norm_ref):
    s = jnp.sum(dp_ref[...], axis=0, keepdims=True)
    norm_ref[...] = lax.rsqrt(jnp.maximum(s, 1.0))


def _norm_tc(deg_parts):
    nc, n_pad = deg_parts.shape
    return pl.pallas_call(
        _norm_body,
        out_shape=jax.ShapeDtypeStruct((1, n_pad), jnp.float32),
    )(deg_parts)


def _scale_body(h_ref, n_ref, o_ref):
    o_ref[...] = h_ref[...] * n_ref[...]


def _scale_tc(h, norm_col):
    n, d = h.shape
    bn = 2000
    return pl.pallas_call(
        _scale_body,
        grid=(n // bn,),
        in_specs=[pl.BlockSpec((bn, d), lambda i: (i, 0)),
                  pl.BlockSpec((bn, 1), lambda i: (i, 0))],
        out_specs=pl.BlockSpec((bn, d), lambda i: (i, 0)),
        out_shape=jax.ShapeDtypeStruct((n, d), jnp.float32),
    )(h, norm_col)


def _spmm_sc(hn, col, row, n_pad):
    n, d = hn.shape
    e = col.shape[0]
    tch = e // _CH
    rpt = n_pad // _NS
    zrows = _CH
    assert tch * _CH == e and rpt % zrows == 0 and d % _L == 0

    @functools.partial(
        pl.kernel,
        out_type=jax.ShapeDtypeStruct((_NC, n_pad, d), jnp.float32),
        mesh=_sc_mesh(),
        scratch_types=[
            pltpu.VMEM((_CH,), jnp.int32),
            pltpu.VMEM((_CH,), jnp.int32),
            pltpu.VMEM((_CH, d), jnp.float32),
            pltpu.VMEM((zrows, d), jnp.float32),
            pltpu.VMEM_SHARED((n_pad, d), jnp.float32),
        ],
    )
    def spmm_kernel(hn_hbm, col_hbm, row_hbm, out_hbm,
                    cidx, ridx, gbuf, zbuf, agg_s):
        c = lax.axis_index("c")
        s = lax.axis_index("s")
        w = c * _NS + s
        lo = w * tch // _NW
        hi = (w + 1) * tch // _NW
        zeros = jnp.zeros((_L,), jnp.float32)

        def zrow(r, carry):
            for q in range(d // _L):
                zbuf[r, pl.ds(q * _L, _L)] = zeros
            return carry
        lax.fori_loop(0, zrows, zrow, None)

        for k in range(rpt // zrows):
            pltpu.sync_copy(zbuf, agg_s.at[pl.ds(s * rpt + k * zrows, zrows)])
        plsc.subcore_barrier()

        def body(j, carry):
            off = j * _CH
            pltpu.sync_copy(col_hbm.at[pl.ds(off, _CH)], cidx)
            pltpu.sync_copy(hn_hbm.at[cidx], gbuf)
            pltpu.sync_copy(row_hbm.at[pl.ds(off, _CH)], ridx)
            pltpu.sync_copy(gbuf, agg_s.at[ridx], add=True)
            return carry
        lax.fori_loop(lo, hi, body, None)

        plsc.subcore_barrier()
        pltpu.sync_copy(agg_s.at[pl.ds(s * rpt, rpt)],
                        out_hbm.at[c].at[pl.ds(s * rpt, rpt)])

    return spmm_kernel(hn, col, row)


def _final_body(a_ref, n_ref, w_ref, o_ref):
    a = a_ref[0] + a_ref[1]
    sc = a * n_ref[...]
    o_ref[...] = lax.dot_general(
        sc, w_ref[...], (((1,), (1,)), ((), ())),
        preferred_element_type=jnp.float32)


def _final_tc(agg2, norm_col, W):
    _, n_pad, d = agg2.shape
    bn = 2048
    assert n_pad % bn == 0
    return pl.pallas_call(
        _final_body,
        grid=(n_pad // bn,),
        in_specs=[pl.BlockSpec((2, bn, d), lambda i: (0, i, 0)),
                  pl.BlockSpec((bn, 1), lambda i: (i, 0)),
                  pl.BlockSpec((d, d), lambda i: (0, 0))],
        out_specs=pl.BlockSpec((bn, d), lambda i: (i, 0)),
        out_shape=jax.ShapeDtypeStruct((n_pad, d), jnp.float32),
    )(agg2, norm_col, W)


def kernel(edge_index, h, W):
    n, d = h.shape
    n_pad = _pad_n(n)
    row = edge_index[0]
    col = edge_index[1]
    deg_parts = _deg_sc(row, n_pad)
    norm = _norm_tc(deg_parts)
    norm_col = norm.reshape(n_pad, 1)
    hn = _scale_tc(h, norm_col[:n])
    agg2 = _spmm_sc(hn, col, row, n_pad)
    out = _final_tc(agg2, norm_col, W)
    return out[:n]
